# baseline (speedup 1.0000x reference)
import functools

import jax
import jax.numpy as jnp
from jax import lax
from jax.experimental import pallas as pl
from jax.experimental.pallas import tpu as pltpu
from jax.experimental.pallas import tpu_sc as plsc

D_MODEL = 768
HIDDEN = 3072
NUM_EXPERTS = 16
NT = 2048
NT_PAD = 2560
LOAD_BALANCE_COEF = 0.01

RBLK = 512
SWIDTH = 128
SLAB = 256
GMAX = 24
_SQRT_HALF = 0.7071067811865476


def _router_body(x_ref, gw_ref, p_ref, scoreb_ref, counts_ref,
                 psum_ref, sel_scr, rank_scr, score_scr, runc, pacc):
    t = pl.program_id(0)
    nb = NT // RBLK

    @pl.when(t == 0)
    def _():
        runc[...] = jnp.zeros_like(runc)
        pacc[...] = jnp.zeros_like(pacc)

    @pl.when(t < nb)
    def _():
        xb = x_ref[...]
        gw = gw_ref[...]
        logits = jnp.dot(xb, gw, preferred_element_type=jnp.float32)
        m = jnp.max(logits, axis=1, keepdims=True)
        ex = jnp.exp(logits - m)
        s = jnp.sum(ex, axis=1, keepdims=True)
        prob = ex / s
        mp = jnp.max(prob, axis=1, keepdims=True)
        eidx = lax.broadcasted_iota(jnp.int32, (RBLK, NUM_EXPERTS), 1)
        sel = jnp.min(jnp.where(prob == mp, eidx, NUM_EXPERTS), axis=1)
        onehot = (eidx == sel[:, None]).astype(jnp.float32)
        r = lax.broadcasted_iota(jnp.int32, (RBLK, RBLK), 0)
        c = lax.broadcasted_iota(jnp.int32, (RBLK, RBLK), 1)
        tri = (r >= c).astype(jnp.float32)
        inc = jnp.dot(tri, onehot, preferred_element_type=jnp.float32)
        rank = jnp.sum(onehot * (runc[...] + inc), axis=1) - 1.0
        runc[...] = runc[...] + jnp.sum(onehot, axis=0, keepdims=True)
        pacc[...] = pacc[...] + jnp.sum(prob, axis=0, keepdims=True)
        sel_scr[pl.ds(t, 1), :] = sel[None, :]
        rank_scr[pl.ds(t, 1), :] = rank[None, :]
        score_scr[pl.ds(t, 1), :] = mp[:, 0][None, :]

    @pl.when(t == nb)
    def _():
        counts_ref[...] = runc[...]
        psum_ref[...] = pacc[...]
        cnt = runc[...]
        cp = jnp.floor((cnt + 7.0) / 8.0) * 8.0
        ii = lax.broadcasted_iota(jnp.int32, (NUM_EXPERTS, NUM_EXPERTS), 0)
        jj = lax.broadcasted_iota(jnp.int32, (NUM_EXPERTS, NUM_EXPERTS), 1)
        offm = jnp.where(jj < ii, jnp.broadcast_to(cp, ii.shape), 0.0)
        off = jnp.sum(offm, axis=1)
        sel_all = sel_scr[...]
        eidx3 = lax.broadcasted_iota(jnp.int32, (nb, RBLK, NUM_EXPERTS), 2)
        oh3 = (sel_all[:, :, None] == eidx3).astype(jnp.float32)
        offsel = jnp.sum(oh3 * off[None, None, :], axis=2)
        p_all = (offsel + rank_scr[...]).astype(jnp.int32)
        p_ref[...] = p_all[:, None, :]
        scoreb_ref[...] = jnp.broadcast_to(
            score_scr[...][:, :, None], (nb, RBLK, SWIDTH))


def _router(x2, gate_W):
    nb = NT // RBLK
    return pl.pallas_call(
        _router_body,
        grid=(nb + 1,),
        in_specs=[
            pl.BlockSpec((RBLK, D_MODEL),
                         lambda t: (jnp.minimum(t, NT // RBLK - 1), 0)),
            pl.BlockSpec((D_MODEL, NUM_EXPERTS), lambda t: (0, 0)),
        ],
        out_specs=[
            pl.BlockSpec((nb, 1, RBLK), lambda t: (0, 0, 0)),
            pl.BlockSpec((nb, RBLK, SWIDTH), lambda t: (0, 0, 0)),
            pl.BlockSpec((1, NUM_EXPERTS), lambda t: (0, 0)),
            pl.BlockSpec((1, NUM_EXPERTS), lambda t: (0, 0)),
        ],
        out_shape=[
            jax.ShapeDtypeStruct((nb, 1, RBLK), jnp.int32),
            jax.ShapeDtypeStruct((nb, RBLK, SWIDTH), jnp.float32),
            jax.ShapeDtypeStruct((1, NUM_EXPERTS), jnp.float32),
            jax.ShapeDtypeStruct((1, NUM_EXPERTS), jnp.float32),
        ],
        scratch_shapes=[
            pltpu.VMEM((nb, RBLK), jnp.int32),
            pltpu.VMEM((nb, RBLK), jnp.float32),
            pltpu.VMEM((nb, RBLK), jnp.float32),
            pltpu.VMEM((1, NUM_EXPERTS), jnp.float32),
            pltpu.VMEM((1, NUM_EXPERTS), jnp.float32),
        ],
        compiler_params=pltpu.CompilerParams(
            dimension_semantics=("arbitrary",)),
    )(x2, gate_W)


def _ffn_body(ex_ref, st_ref, en_ref, xg_ref, ss_ref, w1_ref, b1_ref,
              w2_ref, b2_ref, out_ref):
    g = pl.program_id(0)
    start = pl.multiple_of(st_ref[g], 8)
    end = en_ref[g]

    @pl.when(start < end)
    def _():
        x = xg_ref[pl.ds(start, SLAB), :]
        h = jnp.dot(x, w1_ref[0], preferred_element_type=jnp.float32)
        h = h + b1_ref[0]
        h = 0.5 * h * (1.0 + lax.erf(h * _SQRT_HALF))
        y = jnp.dot(h, w2_ref[0], preferred_element_type=jnp.float32)
        y = y + b2_ref[0]
        y = y * ss_ref[pl.ds(start, SLAB), 0:1]
        rows = lax.broadcasted_iota(jnp.int32, (SLAB, 1), 0) + start
        mask = rows < end
        out_ref[pl.ds(start, SLAB), :] = jnp.where(
            mask, y, out_ref[pl.ds(start, SLAB), :])


def _grouped_ffn(experts_g, starts_g, ends_g, xg, ss, W1, b1, W2, b2):
    grid_spec = pltpu.PrefetchScalarGridSpec(
        num_scalar_prefetch=3,
        grid=(GMAX,),
        in_specs=[
            pl.BlockSpec((NT_PAD, D_MODEL), lambda g, ex, st, en: (0, 0)),
            pl.BlockSpec((NT_PAD, SWIDTH), lambda g, ex, st, en: (0, 0)),
            pl.BlockSpec((1, D_MODEL, HIDDEN),
                         lambda g, ex, st, en: (ex[g], 0, 0)),
            pl.BlockSpec((1, 1, HIDDEN), lambda g, ex, st, en: (ex[g], 0, 0)),
            pl.BlockSpec((1, HIDDEN, D_MODEL),
                         lambda g, ex, st, en: (ex[g], 0, 0)),
            pl.BlockSpec((1, 1, D_MODEL), lambda g, ex, st, en: (ex[g], 0, 0)),
        ],
        out_specs=pl.BlockSpec((NT_PAD, D_MODEL), lambda g, ex, st, en: (0, 0)),
    )
    return pl.pallas_call(
        _ffn_body,
        grid_spec=grid_spec,
        out_shape=jax.ShapeDtypeStruct((NT_PAD, D_MODEL), jnp.float32),
        compiler_params=pltpu.CompilerParams(
            dimension_semantics=("arbitrary",),
            vmem_limit_bytes=128 * 1024 * 1024),
    )(experts_g, starts_g, ends_g, xg, ss,
      W1, b1.reshape(NUM_EXPERTS, 1, HIDDEN), W2,
      b2.reshape(NUM_EXPERTS, 1, D_MODEL))


_NW = 32
_CHUNK = NT // _NW


def _sc_wid():
    return lax.axis_index("s") * 2 + lax.axis_index("c")


@functools.cache
def _sc_kernels():
    mesh = plsc.VectorSubcoreMesh(core_axis_name="c", subcore_axis_name="s")

    @functools.partial(
        pl.kernel, mesh=mesh,
        out_type=[
            jax.ShapeDtypeStruct((NT_PAD, D_MODEL), jnp.float32),
            jax.ShapeDtypeStruct((NT_PAD, SWIDTH), jnp.float32),
        ],
        scratch_types=[
            pltpu.VMEM((_CHUNK,), jnp.int32),
            pltpu.VMEM((_CHUNK, D_MODEL), jnp.float32),
            pltpu.VMEM((_CHUNK, SWIDTH), jnp.float32),
            pltpu.SemaphoreType.DMA,
            pltpu.SemaphoreType.DMA,
        ],
    )
    def sc_dispatch(x_hbm, p_hbm, sc16_hbm, xg_hbm, ss_hbm,
                    idx_v, rows_v, s16_v, sem, sem2):
        base = _sc_wid() * _CHUNK
        pltpu.sync_copy(p_hbm.at[pl.ds(base, _CHUNK)], idx_v)
        pltpu.sync_copy(x_hbm.at[pl.ds(base, _CHUNK)], rows_v)
        pltpu.sync_copy(sc16_hbm.at[pl.ds(base, _CHUNK)], s16_v)
        cp1 = pltpu.async_copy(rows_v, xg_hbm.at[idx_v], sem)
        cp2 = pltpu.async_copy(s16_v, ss_hbm.at[idx_v], sem2)
        cp1.wait()
        cp2.wait()

    @functools.partial(
        pl.kernel, mesh=mesh,
        out_type=jax.ShapeDtypeStruct((NT, D_MODEL), jnp.float32),
        scratch_types=[
            pltpu.VMEM((_CHUNK,), jnp.int32),
            pltpu.VMEM((_CHUNK, D_MODEL), jnp.float32),
            pltpu.SemaphoreType.DMA,
        ],
    )
    def sc_unsort(yg_hbm, p_hbm, out_hbm, idx_v, rows_v, sem):
        base = _sc_wid() * _CHUNK
        pltpu.sync_copy(p_hbm.at[pl.ds(base, _CHUNK)], idx_v)
        pltpu.async_copy(yg_hbm.at[idx_v], rows_v, sem).wait()
        pltpu.sync_copy(rows_v, out_hbm.at[pl.ds(base, _CHUNK)])

    return sc_dispatch, sc_unsort


def kernel(x, gate_W, W1, b1, W2, b2):
    x2 = x.reshape(NT, D_MODEL)
    p3, scoreb, counts2, psum2 = _router(x2, gate_W)
    p = p3.reshape(NT)
    score16 = scoreb.reshape(NT, SWIDTH)
    counts = counts2[0]
    psum = psum2[0]

    c = counts.astype(jnp.int32)
    cpad = (c + 7) // 8 * 8
    offp = jnp.concatenate(
        [jnp.zeros((1,), jnp.int32), jnp.cumsum(cpad)[:-1]])

    n_e = (c + (SLAB - 1)) // SLAB
    cn = jnp.cumsum(n_e)
    j = jnp.arange(GMAX, dtype=jnp.int32)
    e_j = jnp.minimum(
        jnp.searchsorted(cn, j, side="right").astype(jnp.int32),
        NUM_EXPERTS - 1)
    k_j = j - (cn[e_j] - n_e[e_j])
    valid = j < cn[NUM_EXPERTS - 1]
    starts_g = jnp.where(valid, offp[e_j] + SLAB * k_j, 0)
    ends_g = jnp.where(valid, offp[e_j] + c[e_j], 0)
    experts_g = jnp.where(valid, e_j, NUM_EXPERTS - 1)

    sc_dispatch, sc_unsort = _sc_kernels()
    xg, ss16 = sc_dispatch(x2, p, score16)
    yg = _grouped_ffn(experts_g, starts_g, ends_g, xg, ss16, W1, b1, W2, b2)
    out = sc_unsort(yg, p)

    f = counts / NT
    P = psum / NT
    aux_loss = LOAD_BALANCE_COEF * (NUM_EXPERTS * jnp.sum(f * P))
    return out.reshape(1, NT, D_MODEL), aux_loss


# speedup vs baseline: 1.0173x; 1.0173x over previous
import functools

import jax
import jax.numpy as jnp
from jax import lax
from jax.experimental import pallas as pl
from jax.experimental.pallas import tpu as pltpu
from jax.experimental.pallas import tpu_sc as plsc

D_MODEL = 768
HIDDEN = 3072
NUM_EXPERTS = 16
NT = 2048
NT_PAD = 2560
LOAD_BALANCE_COEF = 0.01

RBLK = 256
SWIDTH = 128
SLAB = 256
GMAX = 24
_SQRT_HALF = 0.7071067811865476


def _router_body(x_ref, gw_ref, p_ref, scoreb_ref, counts_ref,
                 psum_ref, sel_scr, rank_scr, score_scr, runc, pacc):
    t = pl.program_id(0)
    nb = NT // RBLK

    @pl.when(t == 0)
    def _():
        runc[...] = jnp.zeros_like(runc)
        pacc[...] = jnp.zeros_like(pacc)

    @pl.when(t < nb)
    def _():
        xb = x_ref[...]
        gw = gw_ref[...]
        logits = jnp.dot(xb, gw, preferred_element_type=jnp.float32)
        m = jnp.max(logits, axis=1, keepdims=True)
        ex = jnp.exp(logits - m)
        s = jnp.sum(ex, axis=1, keepdims=True)
        prob = ex / s
        mp = jnp.max(prob, axis=1, keepdims=True)
        eidx = lax.broadcasted_iota(jnp.int32, (RBLK, NUM_EXPERTS), 1)
        sel = jnp.min(jnp.where(prob == mp, eidx, NUM_EXPERTS), axis=1)
        onehot = (eidx == sel[:, None]).astype(jnp.float32)
        r = lax.broadcasted_iota(jnp.int32, (RBLK, RBLK), 0)
        c = lax.broadcasted_iota(jnp.int32, (RBLK, RBLK), 1)
        tri = (r >= c).astype(jnp.float32)
        inc = jnp.dot(tri, onehot, preferred_element_type=jnp.float32)
        rank = jnp.sum(onehot * (runc[...] + inc), axis=1) - 1.0
        runc[...] = runc[...] + jnp.sum(onehot, axis=0, keepdims=True)
        pacc[...] = pacc[...] + jnp.sum(prob, axis=0, keepdims=True)
        sel_scr[pl.ds(t, 1), :] = sel[None, :]
        rank_scr[pl.ds(t, 1), :] = rank[None, :]
        score_scr[pl.ds(t, 1), :] = mp[:, 0][None, :]

    @pl.when(t == nb)
    def _():
        counts_ref[...] = runc[...]
        psum_ref[...] = pacc[...]
        cnt = runc[...]
        cp = jnp.floor((cnt + 7.0) / 8.0) * 8.0
        ii = lax.broadcasted_iota(jnp.int32, (NUM_EXPERTS, NUM_EXPERTS), 0)
        jj = lax.broadcasted_iota(jnp.int32, (NUM_EXPERTS, NUM_EXPERTS), 1)
        offm = jnp.where(jj < ii, jnp.broadcast_to(cp, ii.shape), 0.0)
        off = jnp.sum(offm, axis=1)
        sel_all = sel_scr[...]
        eidx3 = lax.broadcasted_iota(jnp.int32, (nb, RBLK, NUM_EXPERTS), 2)
        oh3 = (sel_all[:, :, None] == eidx3).astype(jnp.float32)
        offsel = jnp.sum(oh3 * off[None, None, :], axis=2)
        p_all = (offsel + rank_scr[...]).astype(jnp.int32)
        p_ref[...] = p_all[:, None, :]
        scoreb_ref[...] = jnp.broadcast_to(
            score_scr[...][:, :, None], (nb, RBLK, SWIDTH))


def _router(x2, gate_W):
    nb = NT // RBLK
    return pl.pallas_call(
        _router_body,
        grid=(nb + 1,),
        in_specs=[
            pl.BlockSpec((RBLK, D_MODEL), lambda t: (jnp.minimum(t, 7), 0)),
            pl.BlockSpec((D_MODEL, NUM_EXPERTS), lambda t: (0, 0)),
        ],
        out_specs=[
            pl.BlockSpec((nb, 1, RBLK), lambda t: (0, 0, 0)),
            pl.BlockSpec((nb, RBLK, SWIDTH), lambda t: (0, 0, 0)),
            pl.BlockSpec((1, NUM_EXPERTS), lambda t: (0, 0)),
            pl.BlockSpec((1, NUM_EXPERTS), lambda t: (0, 0)),
        ],
        out_shape=[
            jax.ShapeDtypeStruct((nb, 1, RBLK), jnp.int32),
            jax.ShapeDtypeStruct((nb, RBLK, SWIDTH), jnp.float32),
            jax.ShapeDtypeStruct((1, NUM_EXPERTS), jnp.float32),
            jax.ShapeDtypeStruct((1, NUM_EXPERTS), jnp.float32),
        ],
        scratch_shapes=[
            pltpu.VMEM((nb, RBLK), jnp.int32),
            pltpu.VMEM((nb, RBLK), jnp.float32),
            pltpu.VMEM((nb, RBLK), jnp.float32),
            pltpu.VMEM((1, NUM_EXPERTS), jnp.float32),
            pltpu.VMEM((1, NUM_EXPERTS), jnp.float32),
        ],
        compiler_params=pltpu.CompilerParams(
            dimension_semantics=("arbitrary",)),
    )(x2, gate_W)


def _ffn_body(ex_ref, st_ref, en_ref, xg_ref, ss_ref, w1_ref, b1_ref,
              w2_ref, b2_ref, out_ref):
    g = pl.program_id(0)
    start = pl.multiple_of(st_ref[g], 8)
    end = en_ref[g]

    @pl.when(start < end)
    def _():
        x = xg_ref[pl.ds(start, SLAB), :]
        h = jnp.dot(x, w1_ref[0], preferred_element_type=jnp.float32)
        h = h + b1_ref[0]
        h = 0.5 * h * (1.0 + lax.erf(h * _SQRT_HALF))
        y = jnp.dot(h, w2_ref[0], preferred_element_type=jnp.float32)
        y = y + b2_ref[0]
        y = y * ss_ref[pl.ds(start, SLAB), 0:1]
        rows = lax.broadcasted_iota(jnp.int32, (SLAB, 1), 0) + start
        mask = rows < end
        out_ref[pl.ds(start, SLAB), :] = jnp.where(
            mask, y, out_ref[pl.ds(start, SLAB), :])


def _grouped_ffn(experts_g, starts_g, ends_g, xg, ss, W1, b1, W2, b2):
    grid_spec = pltpu.PrefetchScalarGridSpec(
        num_scalar_prefetch=3,
        grid=(GMAX,),
        in_specs=[
            pl.BlockSpec((NT_PAD, D_MODEL), lambda g, ex, st, en: (0, 0)),
            pl.BlockSpec((NT_PAD, SWIDTH), lambda g, ex, st, en: (0, 0)),
            pl.BlockSpec((1, D_MODEL, HIDDEN),
                         lambda g, ex, st, en: (ex[g], 0, 0)),
            pl.BlockSpec((1, 1, HIDDEN), lambda g, ex, st, en: (ex[g], 0, 0)),
            pl.BlockSpec((1, HIDDEN, D_MODEL),
                         lambda g, ex, st, en: (ex[g], 0, 0)),
            pl.BlockSpec((1, 1, D_MODEL), lambda g, ex, st, en: (ex[g], 0, 0)),
        ],
        out_specs=pl.BlockSpec((NT_PAD, D_MODEL), lambda g, ex, st, en: (0, 0)),
    )
    return pl.pallas_call(
        _ffn_body,
        grid_spec=grid_spec,
        out_shape=jax.ShapeDtypeStruct((NT_PAD, D_MODEL), jnp.float32),
        compiler_params=pltpu.CompilerParams(
            dimension_semantics=("arbitrary",),
            vmem_limit_bytes=128 * 1024 * 1024),
    )(experts_g, starts_g, ends_g, xg, ss,
      W1, b1.reshape(NUM_EXPERTS, 1, HIDDEN), W2,
      b2.reshape(NUM_EXPERTS, 1, D_MODEL))


_NW = 32
_CHUNK = NT // _NW


def _sc_wid():
    return lax.axis_index("s") * 2 + lax.axis_index("c")


@functools.cache
def _sc_kernels():
    mesh = plsc.VectorSubcoreMesh(core_axis_name="c", subcore_axis_name="s")

    @functools.partial(
        pl.kernel, mesh=mesh,
        out_type=[
            jax.ShapeDtypeStruct((NT_PAD, D_MODEL), jnp.float32),
            jax.ShapeDtypeStruct((NT_PAD, SWIDTH), jnp.float32),
        ],
        scratch_types=[
            pltpu.VMEM((_CHUNK,), jnp.int32),
            pltpu.VMEM((_CHUNK, D_MODEL), jnp.float32),
            pltpu.VMEM((_CHUNK, SWIDTH), jnp.float32),
            pltpu.SemaphoreType.DMA,
            pltpu.SemaphoreType.DMA,
        ],
    )
    def sc_dispatch(x_hbm, p_hbm, sc16_hbm, xg_hbm, ss_hbm,
                    idx_v, rows_v, s16_v, sem, sem2):
        base = _sc_wid() * _CHUNK
        pltpu.sync_copy(p_hbm.at[pl.ds(base, _CHUNK)], idx_v)
        pltpu.sync_copy(x_hbm.at[pl.ds(base, _CHUNK)], rows_v)
        pltpu.sync_copy(sc16_hbm.at[pl.ds(base, _CHUNK)], s16_v)
        cp1 = pltpu.async_copy(rows_v, xg_hbm.at[idx_v], sem)
        cp2 = pltpu.async_copy(s16_v, ss_hbm.at[idx_v], sem2)
        cp1.wait()
        cp2.wait()

    @functools.partial(
        pl.kernel, mesh=mesh,
        out_type=jax.ShapeDtypeStruct((NT, D_MODEL), jnp.float32),
        scratch_types=[
            pltpu.VMEM((_CHUNK,), jnp.int32),
            pltpu.VMEM((_CHUNK, D_MODEL), jnp.float32),
            pltpu.SemaphoreType.DMA,
        ],
    )
    def sc_unsort(yg_hbm, p_hbm, out_hbm, idx_v, rows_v, sem):
        base = _sc_wid() * _CHUNK
        pltpu.sync_copy(p_hbm.at[pl.ds(base, _CHUNK)], idx_v)
        pltpu.async_copy(yg_hbm.at[idx_v], rows_v, sem).wait()
        pltpu.sync_copy(rows_v, out_hbm.at[pl.ds(base, _CHUNK)])

    return sc_dispatch, sc_unsort


def kernel(x, gate_W, W1, b1, W2, b2):
    x2 = x.reshape(NT, D_MODEL)
    p3, scoreb, counts2, psum2 = _router(x2, gate_W)
    p = p3.reshape(NT)
    score16 = scoreb.reshape(NT, SWIDTH)
    counts = counts2[0]
    psum = psum2[0]

    c = counts.astype(jnp.int32)
    cpad = (c + 7) // 8 * 8
    offp = jnp.concatenate(
        [jnp.zeros((1,), jnp.int32), jnp.cumsum(cpad)[:-1]])

    n_e = (c + (SLAB - 1)) // SLAB
    cn = jnp.cumsum(n_e)
    j = jnp.arange(GMAX, dtype=jnp.int32)
    e_j = jnp.minimum(
        jnp.searchsorted(cn, j, side="right").astype(jnp.int32),
        NUM_EXPERTS - 1)
    k_j = j - (cn[e_j] - n_e[e_j])
    valid = j < cn[NUM_EXPERTS - 1]
    starts_g = jnp.where(valid, offp[e_j] + SLAB * k_j, 0)
    ends_g = jnp.where(valid, offp[e_j] + c[e_j], 0)
    experts_g = jnp.where(valid, e_j, NUM_EXPERTS - 1)

    sc_dispatch, sc_unsort = _sc_kernels()
    xg, ss16 = sc_dispatch(x2, p, score16)
    yg = _grouped_ffn(experts_g, starts_g, ends_g, xg, ss16, W1, b1, W2, b2)
    out = sc_unsort(yg, p)

    f = counts / NT
    P = psum / NT
    aux_loss = LOAD_BALANCE_COEF * (NUM_EXPERTS * jnp.sum(f * P))
    return out.reshape(1, NT, D_MODEL), aux_loss
